# baseline (speedup 1.0000x reference)
import functools

import jax
import jax.numpy as jnp
from jax import lax
from jax.experimental import pallas as pl
from jax.experimental.pallas import tpu as pltpu
from jax.experimental.pallas import tpu_sc as plsc

B = 64
N = 50000
C = 100000
L = 4
CONV = 0.001
INV_C = 1.0 / C

NT = 16
K = 128
NCH = 49
CP = K * NT * NCH
NTC = CP // K
ZC = 400
FK = 40


def _sc_call(t01, tbf, sx, w3, satidx, ones_h):
    mesh = plsc.VectorSubcoreMesh(core_axis_name="c", subcore_axis_name="s")
    f32 = jnp.float32
    bf16 = jnp.bfloat16

    scratch = [
        pltpu.VMEM_SHARED((N + 8, 32), bf16),
        pltpu.VMEM_SHARED((N + 8, 32), bf16),
        pltpu.VMEM_SHARED((N + 40,), f32),
        pltpu.VMEM((4 * K, 32), bf16),
        pltpu.VMEM((4 * K, 32), bf16),
        pltpu.VMEM((ZC,), f32),
        pltpu.VMEM((4 * K,), f32),
        pltpu.VMEM((4 * K,), jnp.int32),
        pltpu.VMEM((1, 32), f32),
        pltpu.VMEM((32,), f32),
        pltpu.VMEM((32,), jnp.int32),
        pltpu.VMEM((FK, 32), bf16),
        pltpu.VMEM((FK + 16,), f32),
        pltpu.VMEM((FK, 32), f32),
        pltpu.VMEM((FK, 32), f32),
    ]

    @functools.partial(
        pl.kernel,
        out_type=jax.ShapeDtypeStruct((2 * N, 32), f32),
        mesh=mesh,
        compiler_params=pltpu.CompilerParams(
            use_tc_tiling_on_sc=False, needs_layout_passes=False),
        scratch_types=scratch,
    )
    def body(t01_h, tbf_h, sx_h, w_h, satidx_h, ones_h_, out_h,
             t_sh, acc_sh, cnt_sh, rbuf, act, zcnt_v, ones_v, sxv,
             wv, satv32, satidx_v, accv, cntv, tv, outv):
        c = lax.axis_index("c")
        s = lax.axis_index("s")
        z16 = jnp.zeros((16,), f32)
        zb32 = jnp.zeros((32,), bf16)

        def zf(k, _):
            act[k, pl.ds(0, 32)] = zb32
            return 0

        lax.fori_loop(0, 4 * K, zf, 0)

        def zf2(i, _):
            zcnt_v[pl.ds(i * 16, 16)] = z16
            return 0

        lax.fori_loop(0, ZC // 16, zf2, 0)

        pltpu.sync_copy(ones_h_, ones_v)
        pltpu.sync_copy(w_h.at[c], wv)
        pltpu.sync_copy(satidx_h, satidx_v)

        NZC = N // ZC

        def zbody(j, _):
            m = s + j * NT
            pltpu.sync_copy(act.at[pl.ds(0, ZC)], acc_sh.at[pl.ds(m * ZC, ZC)])
            pltpu.sync_copy(zcnt_v, cnt_sh.at[pl.ds(m * ZC, ZC)])
            pltpu.sync_copy(tbf_h.at[pl.ds(c * N + m * ZC, ZC)],
                            rbuf.at[pl.ds(0, ZC)])
            pltpu.sync_copy(rbuf.at[pl.ds(0, ZC)], t_sh.at[pl.ds(m * ZC, ZC)])
            return 0

        nz_mine = jnp.where(s < NZC - (NZC // NT) * NT, NZC // NT + 1, NZC // NT)
        lax.fori_loop(0, nz_mine, zbody, 0)

        @pl.when(s == 0)
        def _():
            pltpu.sync_copy(act.at[pl.ds(0, 8)], acc_sh.at[pl.ds(N, 8)])
            pltpu.sync_copy(act.at[pl.ds(0, 8)], t_sh.at[pl.ds(N, 8)])
            pltpu.sync_copy(zcnt_v.at[pl.ds(0, 40)], cnt_sh.at[pl.ds(N, 40)])

        plsc.subcore_barrier()

        one_bf = jnp.ones((), bf16)
        zero_bf = jnp.zeros((), bf16)

        def chunk_body(j, carry):
            sa0, sa1 = carry
            mm = s * NCH + j
            pltpu.sync_copy(sx_h.at[pl.ds(mm * 512, 4 * K)], sxv)
            pltpu.sync_copy(t_sh.at[sxv], rbuf)

            def kbody(k, kc):
                ka0, ka1 = kc
                sb = (rbuf[k, pl.ds(0, 32)] + rbuf[K + k, pl.ds(0, 32)]
                      + rbuf[2 * K + k, pl.ds(0, 32)]
                      + rbuf[3 * K + k, pl.ds(0, 32)])
                minv = jnp.minimum(sb, one_bf)
                u0, u1 = plsc.unpack(minv, format=plsc.PackFormat.INTERLEAVED)
                av = jnp.where(sb < one_bf, one_bf, zero_bf)
                for l in range(4):
                    act[l * K + k, pl.ds(0, 32)] = av
                return (ka0 + u0, ka1 + u1)

            sa0, sa1 = lax.fori_loop(0, K, kbody, (sa0, sa1), unroll=4)
            pltpu.sync_copy(act, acc_sh.at[sxv], add=True)
            pltpu.sync_copy(ones_v, cnt_sh.at[sxv], add=True)
            return sa0, sa1

        sa0, sa1 = lax.fori_loop(0, NCH, chunk_body, (z16, z16))

        satv32[pl.ds(0, 16)] = sa0
        satv32[pl.ds(16, 16)] = sa1
        pltpu.sync_copy(satv32, cnt_sh.at[satidx_v], add=True)
        plsc.subcore_barrier()
        pltpu.sync_copy(cnt_sh.at[pl.ds(N + 8, 32)], satv32)
        t0s = satv32[pl.ds(0, 16)]
        t1s = satv32[pl.ds(16, 16)]
        sat0 = t0s * INV_C
        sat1 = t1s * INV_C
        w0 = wv[0, pl.ds(0, 16)]
        w1 = wv[0, pl.ds(16, 16)]
        d0 = w0 - sat0
        d1 = w1 - sat1
        dsC0 = jnp.where(jnp.abs(d0) > CONV, d0, 0.0) * INV_C
        dsC1 = jnp.where(jnp.abs(d1) > CONV, d1, 0.0) * INV_C

        NRC = N // FK

        def fin(j, _):
            m = s + j * NT
            rn = m * FK
            pltpu.sync_copy(acc_sh.at[pl.ds(rn, FK)], accv)
            pltpu.sync_copy(cnt_sh.at[pl.ds(rn, FK)], cntv.at[pl.ds(0, FK)])
            pltpu.sync_copy(t01_h.at[pl.ds(c * N + rn, FK)], tv)

            def fb(i, __):
                cwin = cntv[pl.ds(i, 16)]
                cv = jnp.full((16,), cwin[0], f32)
                recip = 1.0 / jnp.maximum(cv, 1.0)
                u0, u1 = plsc.unpack(accv[i, pl.ds(0, 32)],
                                     format=plsc.PackFormat.INTERLEAVED)
                o0 = tv[i, pl.ds(0, 16)] + u0 * dsC0 * recip
                o1 = tv[i, pl.ds(16, 16)] + u1 * dsC1 * recip
                outv[i, pl.ds(0, 16)] = jnp.minimum(jnp.maximum(o0, 0.0), 1.0)
                outv[i, pl.ds(16, 16)] = jnp.minimum(jnp.maximum(o1, 0.0), 1.0)
                return 0

            lax.fori_loop(0, FK, fb, 0)
            pltpu.sync_copy(outv, out_h.at[pl.ds(c * N + rn, FK)])
            return 0

        nrc_mine = jnp.where(s < NRC - (NRC // NT) * NT, NRC // NT + 1, NRC // NT)
        lax.fori_loop(0, nrc_mine, fin, 0)

    return body(t01, tbf, sx, w3, satidx, ones_h)


def kernel(initial_t, w, clause_idx):
    f32 = jnp.float32
    bf16 = jnp.bfloat16
    t3 = initial_t.reshape(2, 32, N).transpose(0, 2, 1).reshape(2 * N, 32)
    t01 = jnp.concatenate([t3, jnp.zeros((8, 32), f32)], axis=0)
    tbf = t01.reshape(2 * N + 8, 2, 16).transpose(0, 2, 1).reshape(
        2 * N + 8, 32).astype(bf16)

    cidx = clause_idx.astype(jnp.int32)
    padv = jnp.zeros((CP - C, L), jnp.int32)
    cip = jnp.concatenate([cidx, padv], axis=0)
    is_pad = (jnp.arange(CP, dtype=jnp.int32) >= C)[:, None]
    spread = (jnp.arange(CP, dtype=jnp.int32) % 8)[:, None]
    sx = jnp.where(is_pad, N + spread, cip).reshape(
        NTC, K, L).transpose(0, 2, 1).reshape(-1)

    ones512 = jnp.ones((4 * K,), f32)
    w3 = w.astype(f32).reshape(2, 1, 32)
    satidx = N + 8 + jnp.arange(32, dtype=jnp.int32)

    out01 = _sc_call(t01, tbf, sx, w3, satidx, ones512)
    new_t = out01.reshape(2, N, 32).transpose(0, 2, 1).reshape(B, N)
    return jnp.stack([initial_t, new_t])


# speedup vs baseline: 1.0073x; 1.0073x over previous
import functools

import jax
import jax.numpy as jnp
from jax import lax
from jax.experimental import pallas as pl
from jax.experimental.pallas import tpu as pltpu
from jax.experimental.pallas import tpu_sc as plsc

B = 64
N = 50000
C = 100000
L = 4
CONV = 0.001
INV_C = 1.0 / C

NT = 16
K = 112
NCH = 56
CP = K * NT * NCH
NTC = CP // K
ZC = 200
FK = 40


def _sc_call(t01, tbf, sx, w3, satidx, ones_h):
    mesh = plsc.VectorSubcoreMesh(core_axis_name="c", subcore_axis_name="s")
    f32 = jnp.float32
    bf16 = jnp.bfloat16

    scratch = [
        pltpu.VMEM_SHARED((N + 8, 32), bf16),
        pltpu.VMEM_SHARED((N + 8, 32), bf16),
        pltpu.VMEM_SHARED((N + 40,), f32),
        pltpu.VMEM((4 * K, 32), bf16),
        pltpu.VMEM((4 * K, 32), bf16),
        pltpu.VMEM((4 * K, 32), bf16),
        pltpu.VMEM((ZC + 8,), f32),
        pltpu.VMEM((4 * K,), f32),
        pltpu.VMEM((4 * K,), jnp.int32),
        pltpu.VMEM((4 * K,), jnp.int32),
        pltpu.VMEM((1, 32), f32),
        pltpu.VMEM((32,), f32),
        pltpu.VMEM((32,), jnp.int32),
        pltpu.VMEM((FK, 32), bf16),
        pltpu.VMEM((FK + 16,), f32),
        pltpu.VMEM((FK, 32), f32),
        pltpu.VMEM((FK, 32), f32),
        pltpu.SemaphoreType.DMA,
        pltpu.SemaphoreType.DMA,
    ]

    @functools.partial(
        pl.kernel,
        out_type=jax.ShapeDtypeStruct((2 * N, 32), f32),
        mesh=mesh,
        compiler_params=pltpu.CompilerParams(
            use_tc_tiling_on_sc=False, needs_layout_passes=False),
        scratch_types=scratch,
    )
    def body(t01_h, tbf_h, sx_h, w_h, satidx_h, ones_h_, out_h,
             t_sh, acc_sh, cnt_sh, rbuf0, rbuf1, act, zcnt_v, ones_v,
             sxv0, sxv1, wv, satv32, satidx_v, accv, cntv, tv, outv,
             sem_g0, sem_g1):
        rbuf = rbuf0
        rbufs = (rbuf0, rbuf1)
        sxvs = (sxv0, sxv1)
        sem_g = (sem_g0, sem_g1)
        c = lax.axis_index("c")
        s = lax.axis_index("s")
        z16 = jnp.zeros((16,), f32)
        zb32 = jnp.zeros((32,), bf16)

        def zf(k, _):
            act[k, pl.ds(0, 32)] = zb32
            return 0

        lax.fori_loop(0, 4 * K, zf, 0)

        def zf2(i, _):
            zcnt_v[pl.ds(i * 16, 16)] = z16
            return 0

        lax.fori_loop(0, (ZC + 8) // 16, zf2, 0)

        pltpu.sync_copy(ones_h_, ones_v)
        pltpu.sync_copy(w_h.at[c], wv)
        pltpu.sync_copy(satidx_h, satidx_v)

        NZC = N // ZC

        def zbody(j, _):
            m = s + j * NT
            pltpu.sync_copy(act.at[pl.ds(0, ZC)], acc_sh.at[pl.ds(m * ZC, ZC)])
            pltpu.sync_copy(zcnt_v.at[pl.ds(0, ZC)],
                            cnt_sh.at[pl.ds(m * ZC, ZC)])
            pltpu.sync_copy(tbf_h.at[pl.ds(c * N + m * ZC, ZC)],
                            rbuf.at[pl.ds(0, ZC)])
            pltpu.sync_copy(rbuf.at[pl.ds(0, ZC)], t_sh.at[pl.ds(m * ZC, ZC)])
            return 0

        nz_mine = jnp.where(s < NZC - (NZC // NT) * NT, NZC // NT + 1, NZC // NT)
        lax.fori_loop(0, nz_mine, zbody, 0)

        @pl.when(s == 0)
        def _():
            pltpu.sync_copy(act.at[pl.ds(0, 8)], acc_sh.at[pl.ds(N, 8)])
            pltpu.sync_copy(act.at[pl.ds(0, 8)], t_sh.at[pl.ds(N, 8)])
            pltpu.sync_copy(zcnt_v.at[pl.ds(0, 40)], cnt_sh.at[pl.ds(N, 40)])

        plsc.subcore_barrier()

        one_bf = jnp.ones((), bf16)
        zero_bf = jnp.zeros((), bf16)

        def compute_chunk(p, sa0, sa1):
            rb = rbufs[p]

            def kbody(k, kc):
                ka0, ka1 = kc
                sb = (rb[k, pl.ds(0, 32)] + rb[K + k, pl.ds(0, 32)]
                      + rb[2 * K + k, pl.ds(0, 32)]
                      + rb[3 * K + k, pl.ds(0, 32)])
                minv = jnp.minimum(sb, one_bf)
                u0, u1 = plsc.unpack(minv, format=plsc.PackFormat.INTERLEAVED)
                av = jnp.where(sb < one_bf, one_bf, zero_bf)
                for l in range(4):
                    act[l * K + k, pl.ds(0, 32)] = av
                return (ka0 + u0, ka1 + u1)

            return lax.fori_loop(0, K, kbody, (sa0, sa1), unroll=4)

        lb = s * NCH
        pltpu.sync_copy(sx_h.at[pl.ds(lb * 4 * K, 4 * K)], sxv0)
        pltpu.async_copy(t_sh.at[sxv0], rbuf0, sem_g0)

        def chunk_pair(j2, carry):
            sa0, sa1 = carry
            for p in range(2):
                m = 2 * j2 + p
                pltpu.make_async_copy(t_sh.at[sxvs[p]], rbufs[p],
                                      sem_g[p]).wait()

                @pl.when(m + 1 < NCH)
                def _(m=m, p=p):
                    pltpu.sync_copy(
                        sx_h.at[pl.ds((lb + m + 1) * 4 * K, 4 * K)],
                        sxvs[1 - p])
                    pltpu.async_copy(t_sh.at[sxvs[1 - p]], rbufs[1 - p],
                                     sem_g[1 - p])

                sa0, sa1 = compute_chunk(p, sa0, sa1)
                pltpu.sync_copy(act, acc_sh.at[sxvs[p]], add=True)
                pltpu.sync_copy(ones_v, cnt_sh.at[sxvs[p]], add=True)
            return sa0, sa1

        sa0, sa1 = lax.fori_loop(0, NCH // 2, chunk_pair, (z16, z16))

        satv32[pl.ds(0, 16)] = sa0
        satv32[pl.ds(16, 16)] = sa1
        pltpu.sync_copy(satv32, cnt_sh.at[satidx_v], add=True)
        plsc.subcore_barrier()
        pltpu.sync_copy(cnt_sh.at[pl.ds(N + 8, 32)], satv32)
        t0s = satv32[pl.ds(0, 16)]
        t1s = satv32[pl.ds(16, 16)]
        sat0 = t0s * INV_C
        sat1 = t1s * INV_C
        w0 = wv[0, pl.ds(0, 16)]
        w1 = wv[0, pl.ds(16, 16)]
        d0 = w0 - sat0
        d1 = w1 - sat1
        dsC0 = jnp.where(jnp.abs(d0) > CONV, d0, 0.0) * INV_C
        dsC1 = jnp.where(jnp.abs(d1) > CONV, d1, 0.0) * INV_C

        NRC = N // FK

        def fin(j, _):
            m = s + j * NT
            rn = m * FK
            pltpu.sync_copy(acc_sh.at[pl.ds(rn, FK)], accv)
            pltpu.sync_copy(cnt_sh.at[pl.ds(rn, FK)], cntv.at[pl.ds(0, FK)])
            pltpu.sync_copy(t01_h.at[pl.ds(c * N + rn, FK)], tv)

            def fb(i, __):
                cwin = cntv[pl.ds(i, 16)]
                cv = jnp.full((16,), cwin[0], f32)
                recip = 1.0 / jnp.maximum(cv, 1.0)
                u0, u1 = plsc.unpack(accv[i, pl.ds(0, 32)],
                                     format=plsc.PackFormat.INTERLEAVED)
                o0 = tv[i, pl.ds(0, 16)] + u0 * dsC0 * recip
                o1 = tv[i, pl.ds(16, 16)] + u1 * dsC1 * recip
                outv[i, pl.ds(0, 16)] = jnp.minimum(jnp.maximum(o0, 0.0), 1.0)
                outv[i, pl.ds(16, 16)] = jnp.minimum(jnp.maximum(o1, 0.0), 1.0)
                return 0

            lax.fori_loop(0, FK, fb, 0)
            pltpu.sync_copy(outv, out_h.at[pl.ds(c * N + rn, FK)])
            return 0

        nrc_mine = jnp.where(s < NRC - (NRC // NT) * NT, NRC // NT + 1, NRC // NT)
        lax.fori_loop(0, nrc_mine, fin, 0)

    return body(t01, tbf, sx, w3, satidx, ones_h)


def kernel(initial_t, w, clause_idx):
    f32 = jnp.float32
    bf16 = jnp.bfloat16
    t3 = initial_t.reshape(2, 32, N).transpose(0, 2, 1).reshape(2 * N, 32)
    t01 = jnp.concatenate([t3, jnp.zeros((8, 32), f32)], axis=0)
    tbf = t01.reshape(2 * N + 8, 2, 16).transpose(0, 2, 1).reshape(
        2 * N + 8, 32).astype(bf16)

    cidx = clause_idx.astype(jnp.int32)
    padv = jnp.zeros((CP - C, L), jnp.int32)
    cip = jnp.concatenate([cidx, padv], axis=0)
    is_pad = (jnp.arange(CP, dtype=jnp.int32) >= C)[:, None]
    spread = (jnp.arange(CP, dtype=jnp.int32) % 8)[:, None]
    sx = jnp.where(is_pad, N + spread, cip).reshape(
        NTC, K, L).transpose(0, 2, 1).reshape(-1)

    ones512 = jnp.ones((4 * K,), f32)
    w3 = w.astype(f32).reshape(2, 1, 32)
    satidx = N + 8 + jnp.arange(32, dtype=jnp.int32)

    out01 = _sc_call(t01, tbf, sx, w3, satidx, ones512)
    new_t = out01.reshape(2, N, 32).transpose(0, 2, 1).reshape(B, N)
    return jnp.stack([initial_t, new_t])
